# baseline (speedup 1.0000x reference)
import functools

import jax
import jax.numpy as jnp
from jax.experimental import pallas as pl

N, E, D, DE, H, G = 10000, 320000, 128, 16, 16, 64

_BE = 1000


def _act_body(zf_ref, zs_ref, m_ref):
    zf = zf_ref[...]
    zs = zs_ref[...]
    sig = 1.0 / (1.0 + jnp.exp(-zf))
    sp = jnp.maximum(zs, 0.0) + jnp.log1p(jnp.exp(-jnp.abs(zs)))
    m_ref[...] = sig * sp


def _act(zf, zs):
    grid = (E // _BE,)
    return pl.pallas_call(
        _act_body,
        grid=grid,
        in_specs=[
            pl.BlockSpec((_BE, D), lambda i: (i, 0)),
            pl.BlockSpec((_BE, D), lambda i: (i, 0)),
        ],
        out_specs=pl.BlockSpec((_BE, D), lambda i: (i, 0)),
        out_shape=jax.ShapeDtypeStruct((E, D), jnp.float32),
    )(zf, zs)


def _layer(h, src, dst, ep_f, ep_s, Wf, bf, Ws, bs):
    td = h @ jnp.concatenate([Wf[:D], Ws[:D]], axis=1) + jnp.concatenate([bf, bs])
    us = h @ jnp.concatenate([Wf[D:2 * D], Ws[D:2 * D]], axis=1)
    zcat = td[dst] + us[src]
    zf = zcat[:, :D] + ep_f
    zs = zcat[:, D:] + ep_s
    m = _act(zf, zs)
    return h + jnp.zeros_like(h).at[dst].add(m)


def kernel(x, edge_index, edge_attr, batch, Wf1, bf1, Ws1, bs1,
           Wf2, bf2, Ws2, bs2, W1, b1, W2, b2):
    src, dst = edge_index[0], edge_index[1]
    ep = edge_attr @ jnp.concatenate(
        [Wf1[2 * D:], Ws1[2 * D:], Wf2[2 * D:], Ws2[2 * D:]], axis=1)
    h = _layer(x, src, dst, ep[:, :D], ep[:, D:2 * D], Wf1, bf1, Ws1, bs1)
    h = _layer(h, src, dst, ep[:, 2 * D:3 * D], ep[:, 3 * D:], Wf2, bf2, Ws2, bs2)
    sums = jax.ops.segment_sum(h, batch, num_segments=G)
    counts = jax.ops.segment_sum(jnp.ones((N, 1), dtype=h.dtype), batch,
                                 num_segments=G)
    pooled = sums / jnp.maximum(counts, 1.0)
    return jax.nn.relu(pooled @ W1 + b1) @ W2 + b2
